# baseline (speedup 1.0000x reference)
import functools

import jax
import jax.numpy as jnp
from jax import lax
from jax.experimental import pallas as pl
from jax.experimental.pallas import tpu as pltpu
from jax.experimental.pallas import tpu_sc as plsc

B, H, N_HI, C = 1, 4, 163842, 32
N_LO, K = 40962, 7
HC = H * C

NC, NS, L = 2, 16, 16
NW = NC * NS
CH = 64
CHT = 66
NIDX = CHT * K
NIDXP = 464
NCHUNK = 640
PER_W = NCHUNK // NW
IDX_PAD_LEN = (NCHUNK - 1) * CH * K + NIDXP
NT_HI = 1281

_mesh = plsc.VectorSubcoreMesh(core_axis_name="c", subcore_axis_name="s")


def _table_body(x_ref, tab_ref):
    blk = x_ref[0]
    blk = jnp.reshape(blk, (HC, 128))
    tab_ref[...] = jnp.transpose(blk)


def _build_table(xt):
    return pl.pallas_call(
        _table_body,
        grid=(NT_HI,),
        in_specs=[pl.BlockSpec((1, H, C, 128), lambda j: (0, 0, 0, j))],
        out_specs=pl.BlockSpec((128, HC), lambda j: (j, 0)),
        out_shape=jax.ShapeDtypeStruct((N_HI, HC), jnp.float32),
    )(xt)


@functools.partial(
    pl.kernel,
    mesh=_mesh,
    out_type=jax.ShapeDtypeStruct((H * N_LO * C,), jnp.float32),
    scratch_types=[
        pltpu.VMEM((NIDXP,), jnp.int32),
        pltpu.VMEM((NIDXP, HC), jnp.float32),
        pltpu.VMEM((CHT * H * C,), jnp.float32),
        pltpu.SemaphoreType.DMA,
    ],
)
def _hex_pool(tab_hbm, idx_hbm, out_hbm, idx_v, rows_v, out_v, sem):
    wid = lax.axis_index("s") * NC + lax.axis_index("c")

    def chunk_body(t, carry):
        j = wid * PER_W + t

        pltpu.sync_copy(idx_hbm.at[pl.ds(j * CH * K, NIDXP)], idx_v)

        pltpu.async_copy(tab_hbm.at[idx_v], rows_v, sem).wait()

        def row_body(i, carry2):
            r = i * K
            for h in range(H):
                for q in range(2):
                    col = h * C + q * L
                    a = rows_v[r, pl.ds(col, L)]
                    for k in range(1, K):
                        a = jnp.maximum(a, rows_v[r + k, pl.ds(col, L)])
                    out_v[pl.ds((h * CHT + i) * C + q * L, L)] = a
            return carry2

        lax.fori_loop(0, CHT, row_body, 0)

        for h in range(H):
            pltpu.sync_copy(
                out_v.at[pl.ds(h * CHT * C, CH * C)],
                out_hbm.at[pl.ds(h * N_LO * C + j * CH * C, CH * C)])

        @pl.when(j == NCHUNK - 1)
        def _tail():
            for h in range(H):
                pltpu.sync_copy(
                    out_v.at[pl.ds(h * CHT * C + CH * C, (CHT - CH) * C)],
                    out_hbm.at[pl.ds(h * N_LO * C + j * CH * C + CH * C,
                                     (CHT - CH) * C)])

        return carry

    lax.fori_loop(0, PER_W, chunk_body, 0)


def kernel(x, neigh_indices):
    xt = jnp.transpose(x, (0, 1, 3, 2))
    tab = _build_table(xt)
    nf = neigh_indices.astype(jnp.int32).reshape(-1)
    nf = jnp.pad(nf, (0, IDX_PAD_LEN - nf.shape[0]))
    out = _hex_pool(tab, nf)
    return out.reshape(B, H, N_LO, C)


# speedup vs baseline: 1.8975x; 1.8975x over previous
import functools

import jax
import jax.numpy as jnp
from jax import lax
from jax.experimental import pallas as pl
from jax.experimental.pallas import tpu as pltpu
from jax.experimental.pallas import tpu_sc as plsc

B, H, N_HI, C = 1, 4, 163842, 32
N_LO, K = 40962, 7
HC = H * C

NC, NS, L = 2, 16, 16
NW = NC * NS
CH = 64
CHT = 66
NIDX = CHT * K
NIDXP = 464
NCHUNK = 640
PER_W = NCHUNK // NW
IDX_PAD_LEN = (NCHUNK - 1) * CH * K + NIDXP
VBLK = 1024
NT_HI = (N_HI + VBLK - 1) // VBLK

_mesh = plsc.VectorSubcoreMesh(core_axis_name="c", subcore_axis_name="s")


def _table_body(x_ref, tab_ref):
    blk = x_ref[0]
    blk = jnp.reshape(blk, (HC, VBLK))
    tab_ref[...] = jnp.transpose(blk)


def _build_table(xt):
    return pl.pallas_call(
        _table_body,
        grid=(NT_HI,),
        in_specs=[pl.BlockSpec((1, H, C, VBLK), lambda j: (0, 0, 0, j))],
        out_specs=pl.BlockSpec((VBLK, HC), lambda j: (j, 0)),
        out_shape=jax.ShapeDtypeStruct((N_HI, HC), jnp.float32),
    )(xt)


@functools.partial(
    pl.kernel,
    mesh=_mesh,
    out_type=jax.ShapeDtypeStruct((H * N_LO * C,), jnp.float32),
    scratch_types=[
        pltpu.VMEM((NIDXP,), jnp.int32),
        pltpu.VMEM((NIDXP, HC), jnp.float32),
        pltpu.VMEM((CHT * H * C,), jnp.float32),
        pltpu.SemaphoreType.DMA,
    ],
)
def _hex_pool(tab_hbm, idx_hbm, out_hbm, idx_v, rows_v, out_v, sem):
    wid = lax.axis_index("s") * NC + lax.axis_index("c")

    def chunk_body(t, carry):
        j = wid * PER_W + t

        pltpu.sync_copy(idx_hbm.at[pl.ds(j * CH * K, NIDXP)], idx_v)

        pltpu.async_copy(tab_hbm.at[idx_v], rows_v, sem).wait()

        def row_body(i, carry2):
            r = i * K
            for h in range(H):
                for q in range(2):
                    col = h * C + q * L
                    a = rows_v[r, pl.ds(col, L)]
                    for k in range(1, K):
                        a = jnp.maximum(a, rows_v[r + k, pl.ds(col, L)])
                    out_v[pl.ds((h * CHT + i) * C + q * L, L)] = a
            return carry2

        lax.fori_loop(0, CHT, row_body, 0)

        for h in range(H):
            pltpu.sync_copy(
                out_v.at[pl.ds(h * CHT * C, CH * C)],
                out_hbm.at[pl.ds(h * N_LO * C + j * CH * C, CH * C)])

        @pl.when(j == NCHUNK - 1)
        def _tail():
            for h in range(H):
                pltpu.sync_copy(
                    out_v.at[pl.ds(h * CHT * C + CH * C, (CHT - CH) * C)],
                    out_hbm.at[pl.ds(h * N_LO * C + j * CH * C + CH * C,
                                     (CHT - CH) * C)])

        return carry

    lax.fori_loop(0, PER_W, chunk_body, 0)


def kernel(x, neigh_indices):
    xt = jnp.transpose(x, (0, 1, 3, 2))
    tab = _build_table(xt)
    nf = neigh_indices.astype(jnp.int32).reshape(-1)
    nf = jnp.pad(nf, (0, IDX_PAD_LEN - nf.shape[0]))
    out = _hex_pool(tab, nf)
    return out.reshape(B, H, N_LO, C)


# speedup vs baseline: 2.6952x; 1.4204x over previous
import functools

import jax
import jax.numpy as jnp
from jax import lax
from jax.experimental import pallas as pl
from jax.experimental.pallas import tpu as pltpu
from jax.experimental.pallas import tpu_sc as plsc

B, H, N_HI, C = 1, 4, 163842, 32
N_LO, K = 40962, 7
HC = H * C

NC, NS, L = 2, 16, 16
NW = NC * NS
CH = 128
NIDXP = CH * K
NCHUNK = 321
LOOPS_W = 11
IDX_PAD_LEN = NCHUNK * NIDXP
TAIL = N_LO - (NCHUNK - 1) * CH
VBLK = 1024
NT_HI = (N_HI + VBLK - 1) // VBLK

_mesh = plsc.VectorSubcoreMesh(core_axis_name="c", subcore_axis_name="s")


def _table_body(x_ref, tab_ref):
    blk = x_ref[0]
    blk = jnp.reshape(blk, (HC, VBLK))
    tab_ref[...] = jnp.transpose(blk)


def _build_table(xt):
    return pl.pallas_call(
        _table_body,
        grid=(NT_HI,),
        in_specs=[pl.BlockSpec((1, H, C, VBLK), lambda j: (0, 0, 0, j))],
        out_specs=pl.BlockSpec((VBLK, HC), lambda j: (j, 0)),
        out_shape=jax.ShapeDtypeStruct((N_HI, HC), jnp.float32),
    )(xt)


@functools.partial(
    pl.kernel,
    mesh=_mesh,
    compiler_params=pltpu.CompilerParams(needs_layout_passes=False),
    out_type=jax.ShapeDtypeStruct((B, H, C, N_LO), jnp.float32),
    scratch_types=[
        pltpu.VMEM((NIDXP,), jnp.int32),
        pltpu.VMEM((NIDXP, HC), jnp.float32),
        pltpu.VMEM((C, CH), jnp.float32),
        pltpu.SemaphoreType.DMA,
    ],
)
def _hex_pool(tab_hbm, idx_hbm, out_hbm, idx_v, rows_v, out_t, sem):
    wid = lax.axis_index("s") * NC + lax.axis_index("c")
    iota = lax.iota(jnp.int32, L)
    cvec = [iota + q * L for q in range(2)]

    def chunk_body(t, carry):
        j = t * NW + wid

        @pl.when(j < NCHUNK)
        def _chunk():
            pltpu.sync_copy(idx_hbm.at[pl.ds(j * NIDXP, NIDXP)], idx_v)

            pltpu.async_copy(tab_hbm.at[idx_v], rows_v, sem).wait()

            for h in range(H):

                def row_body(i, carry2, h=h):
                    r = i * K
                    vvec = jnp.full((L,), i, jnp.int32)
                    for q in range(2):
                        col = h * C + q * L
                        a = rows_v[r, pl.ds(col, L)]
                        for k in range(1, K):
                            a = jnp.maximum(a, rows_v[r + k, pl.ds(col, L)])
                        plsc.store_scatter(out_t, [cvec[q], vvec], a)
                    return carry2

                lax.fori_loop(0, CH, row_body, 0)

                @pl.when(j < NCHUNK - 1)
                def _full(h=h):
                    pltpu.sync_copy(out_t,
                                    out_hbm.at[0, h, :, pl.ds(j * CH, CH)])

                @pl.when(j == NCHUNK - 1)
                def _tail(h=h):
                    for c in range(C):
                        pltpu.sync_copy(
                            out_t.at[c, pl.ds(0, TAIL)],
                            out_hbm.at[0, h, c, pl.ds((NCHUNK - 1) * CH,
                                                      TAIL)])

        return carry

    lax.fori_loop(0, LOOPS_W, chunk_body, 0)


def kernel(x, neigh_indices):
    xt = jnp.transpose(x, (0, 1, 3, 2))
    tab = _build_table(xt)
    nf = neigh_indices.astype(jnp.int32).reshape(-1)
    nf = jnp.pad(nf, (0, IDX_PAD_LEN - nf.shape[0]))
    out = _hex_pool(tab, nf)
    return jnp.transpose(out, (0, 1, 3, 2))
